# baseline (speedup 1.0000x reference)
import jax
import jax.numpy as jnp
from jax.experimental import pallas as pl
from jax.experimental.pallas import tpu as pltpu


def _copy_kernel(table_ref, out_ref, sem):
    copy = pltpu.make_async_copy(table_ref, out_ref, sem)
    copy.start()
    copy.wait()


def kernel(inputs, table):
    seq_len = inputs.shape[-1]
    rows, dim = table.shape
    assert seq_len == rows
    return pl.pallas_call(
        _copy_kernel,
        in_specs=[pl.BlockSpec(memory_space=pl.ANY)],
        out_specs=pl.BlockSpec(memory_space=pl.ANY),
        out_shape=jax.ShapeDtypeStruct((rows, dim), table.dtype),
        scratch_shapes=[pltpu.SemaphoreType.DMA],
    )(table)


# speedup vs baseline: 34.0555x; 34.0555x over previous
import functools

import jax
import jax.numpy as jnp
from jax.experimental import pallas as pl
from jax.experimental.pallas import tpu as pltpu


def _dbuf_copy(table_hbm, out_hbm, vmem, in_sems, out_sems, *, block):
    n = pl.num_programs(0)
    i = pl.program_id(0)

    def in_copy(j, slot):
        return pltpu.make_async_copy(
            table_hbm.at[pl.ds(j * block, block), :], vmem.at[slot],
            in_sems.at[slot])

    def out_copy(j, slot):
        return pltpu.make_async_copy(
            vmem.at[slot], out_hbm.at[pl.ds(j * block, block), :],
            out_sems.at[slot])

    @pl.when(i == 0)
    def _():
        in_copy(0, 0).start()

    @pl.when(i + 1 < n)
    def _():
        @pl.when(i >= 1)
        def _():
            out_copy(i - 1, (i - 1) % 2).wait()

        in_copy(i + 1, (i + 1) % 2).start()

    in_copy(i, i % 2).wait()
    out_copy(i, i % 2).start()

    @pl.when(i == n - 1)
    def _():
        @pl.when(n >= 2)
        def _():
            out_copy(i - 1, (i - 1) % 2).wait()

        out_copy(i, i % 2).wait()


def kernel(inputs, table):
    seq_len = inputs.shape[-1]
    rows, dim = table.shape
    assert seq_len == rows
    block = 512
    n = rows // block
    return pl.pallas_call(
        functools.partial(_dbuf_copy, block=block),
        grid=(n,),
        in_specs=[pl.BlockSpec(memory_space=pl.ANY)],
        out_specs=pl.BlockSpec(memory_space=pl.ANY),
        out_shape=jax.ShapeDtypeStruct((rows, dim), table.dtype),
        scratch_shapes=[
            pltpu.VMEM((2, block, dim), table.dtype),
            pltpu.SemaphoreType.DMA((2,)),
            pltpu.SemaphoreType.DMA((2,)),
        ],
    )(table)
